# baseline (speedup 1.0000x reference)
import functools

import jax
import jax.numpy as jnp
from jax import lax
from jax.experimental import pallas as pl
from jax.experimental.pallas import tpu as pltpu
from jax.experimental.pallas import tpu_sc as plsc

NC = 2
NS = 16
L = 16
NW = NC * NS

C = 5


def _tc_matmul(Xp, W3, n_pad, d_feat, deg, units):
    BN = 512
    nb = n_pad // BN

    def body(x_ref, w_ref, z_ref):
        x = x_ref[...]
        for k in range(deg):
            z_ref[k] = jnp.dot(x, w_ref[k], preferred_element_type=jnp.float32)

    out3 = pl.pallas_call(
        body,
        grid=(nb,),
        in_specs=[
            pl.BlockSpec((BN, d_feat), lambda i: (i, 0)),
            pl.BlockSpec((deg, d_feat, units), lambda i: (0, 0, 0)),
        ],
        out_specs=pl.BlockSpec((deg, BN, units), lambda i: (0, i, 0)),
        out_shape=jax.ShapeDtypeStruct((deg, n_pad, units), jnp.float32),
    )(Xp, W3)
    return out3.reshape(deg * n_pad, units)


def _sc_gather_reduce(Yr, Gp, init, n_pad, kh, k_off, deg, units):
    per_w = n_pad // NW
    n_chunks = per_w // C
    mesh = plsc.VectorSubcoreMesh(core_axis_name="c", subcore_axis_name="s")
    n_acc = units // L
    first = init.ndim == 1

    @functools.partial(
        pl.kernel,
        mesh=mesh,
        out_type=jax.ShapeDtypeStruct((n_pad, units), jnp.float32),
        scratch_types=[
            pltpu.VMEM((per_w, deg), jnp.int32),
            pltpu.VMEM((2, C, L, units), jnp.float32),
            pltpu.VMEM((per_w, units), jnp.float32),
            pltpu.VMEM((units,), jnp.float32),
            pltpu.SemaphoreType.DMA,
            pltpu.SemaphoreType.DMA,
        ],
    )
    def k(y_hbm, g_hbm, init_hbm, out_hbm, g_all, rows_v, out_all, b_v,
          sem_a, sem_b):
        wid = lax.axis_index("s") * NC + lax.axis_index("c")
        base = wid * per_w
        if first:
            pltpu.sync_copy(init_hbm, b_v)
        else:
            pltpu.sync_copy(init_hbm.at[pl.ds(base, per_w)], out_all)
        pltpu.sync_copy(g_hbm.at[pl.ds(base, per_w)], g_all)
        karr = jnp.arange(L, dtype=jnp.int32) * n_pad

        def fire(ci, slot, sem):
            for n in range(C):
                gvec = g_all[ci * C + n, pl.ds(k_off, L)]
                pltpu.async_copy(y_hbm.at[gvec + karr],
                                 rows_v.at[slot, n], sem)

        def drain(slot, sem):
            for j in range(C):
                pltpu.make_async_copy(y_hbm.at[pl.ds(0, L)],
                                      rows_v.at[slot, j], sem).wait()

        def accum(ci, slot):
            def node_body(nn, c2):
                if first:
                    accs = [b_v[pl.ds(cc * L, L)] for cc in range(n_acc)]
                else:
                    accs = [out_all[ci * C + nn, pl.ds(cc * L, L)]
                            for cc in range(n_acc)]
                for r in range(L):
                    for cc in range(n_acc):
                        accs[cc] = accs[cc] + rows_v[slot, nn, r,
                                                     pl.ds(cc * L, L)]
                for cc in range(n_acc):
                    out_all[ci * C + nn, pl.ds(cc * L, L)] = accs[cc]
                return c2

            lax.fori_loop(0, C, node_body, 0)

        fire(0, 0, sem_a)

        def pair_body(p, carry):
            c0 = 2 * p
            fire(c0 + 1, 1, sem_b)
            drain(0, sem_a)
            accum(c0, 0)
            fire((c0 + 2) % n_chunks, 0, sem_a)
            drain(1, sem_b)
            accum(c0 + 1, 1)
            return carry

        lax.fori_loop(0, n_chunks // 2, pair_body, 0)
        drain(0, sem_a)
        pltpu.sync_copy(out_all, out_hbm.at[pl.ds(base, per_w)])

    return k(Yr, Gp, init)


def kernel(X, G, W, b):
    N, D = X.shape
    DEG = G.shape[1]
    U = W.shape[1]
    block = NW * C * 2
    n_pad = -(-N // block) * block
    n_pad = -(-n_pad // 512) * 512

    W3 = W.reshape(DEG, D, U)
    Xp = jnp.pad(X, ((0, n_pad - N), (0, 0)))
    Gp = jnp.pad(G, ((0, n_pad - N), (0, 0)))

    Xb = Xp.astype(jnp.bfloat16)
    W3b = W3.astype(jnp.bfloat16)
    KH = DEG // 2
    Z1 = _tc_matmul(Xb, W3b[:KH], n_pad, D, KH, U)
    Z2 = _tc_matmul(Xb, W3b[KH:], n_pad, D, KH, U)
    p1 = _sc_gather_reduce(Z1, Gp, b, n_pad, KH, 0, DEG, U)
    out = _sc_gather_reduce(Z2, Gp, p1, n_pad, KH, KH, DEG, U)
    return out[:N]


# speedup vs baseline: 1.1643x; 1.1643x over previous
import functools

import jax
import jax.numpy as jnp
from jax import lax
from jax.experimental import pallas as pl
from jax.experimental.pallas import tpu as pltpu
from jax.experimental.pallas import tpu_sc as plsc

NC = 2
NS = 16
L = 16
NW = NC * NS

C = 5


def _tc_matmul(Xp, W3, n_pad, d_feat, deg, units):
    BN = 1000
    nb = n_pad // BN

    def body(x_ref, w_ref, z_ref):
        x = x_ref[...].astype(jnp.bfloat16)
        for k in range(deg):
            z_ref[k] = jnp.dot(x, w_ref[k], preferred_element_type=jnp.float32)

    out3 = pl.pallas_call(
        body,
        grid=(nb,),
        in_specs=[
            pl.BlockSpec((BN, d_feat), lambda i: (i, 0)),
            pl.BlockSpec((deg, d_feat, units), lambda i: (0, 0, 0)),
        ],
        out_specs=pl.BlockSpec((deg, BN, units), lambda i: (0, i, 0)),
        out_shape=jax.ShapeDtypeStruct((deg, n_pad, units), jnp.float32),
    )(Xp, W3)
    return out3.reshape(deg * n_pad, units)


def _sc_gather_reduce(Yr, Gp, b, n_pad, z_stride, deg, units):
    per_w = n_pad // NW
    n_chunks = per_w // C
    mesh = plsc.VectorSubcoreMesh(core_axis_name="c", subcore_axis_name="s")
    n_acc = units // L

    @functools.partial(
        pl.kernel,
        mesh=mesh,
        out_type=jax.ShapeDtypeStruct((n_pad, units), jnp.float32),
        scratch_types=[
            pltpu.VMEM((per_w, deg), jnp.int32),
            pltpu.VMEM((2, 2 * C, L, units), jnp.float32),
            pltpu.VMEM((per_w, units), jnp.float32),
            pltpu.VMEM((units,), jnp.float32),
            pltpu.SemaphoreType.DMA,
            pltpu.SemaphoreType.DMA,
        ],
    )
    def k(y_hbm, g_hbm, b_hbm, out_hbm, g_all, rows_v, out_all, b_v,
          sem_a, sem_b):
        wid = lax.axis_index("s") * NC + lax.axis_index("c")
        base = wid * per_w
        pltpu.sync_copy(b_hbm, b_v)
        pltpu.sync_copy(g_hbm.at[pl.ds(base, per_w)], g_all)
        karr = [(jnp.arange(L, dtype=jnp.int32) + h * L) * z_stride
                for h in (0, 1)]

        def fire(ci, slot, sem):
            for n in range(C):
                for h in range(2):
                    gvec = g_all[ci * C + n, pl.ds(h * L, L)]
                    pltpu.async_copy(y_hbm.at[gvec + karr[h]],
                                     rows_v.at[slot, 2 * n + h], sem)

        def drain(slot, sem):
            for j in range(2 * C):
                pltpu.make_async_copy(y_hbm.at[pl.ds(0, L)],
                                      rows_v.at[slot, j], sem).wait()

        def accum(ci, slot):
            def node_body(nn, c2):
                accs = [b_v[pl.ds(cc * L, L)] for cc in range(n_acc)]
                for h in range(2):
                    d = 2 * nn + h
                    for r in range(L):
                        for cc in range(n_acc):
                            accs[cc] = accs[cc] + rows_v[slot, d, r,
                                                         pl.ds(cc * L, L)]
                for cc in range(n_acc):
                    out_all[ci * C + nn, pl.ds(cc * L, L)] = accs[cc]
                return c2

            lax.fori_loop(0, C, node_body, 0)

        fire(0, 0, sem_a)

        def pair_body(p, carry):
            c0 = 2 * p
            fire(c0 + 1, 1, sem_b)
            drain(0, sem_a)
            accum(c0, 0)
            fire((c0 + 2) % n_chunks, 0, sem_a)
            drain(1, sem_b)
            accum(c0 + 1, 1)
            return carry

        lax.fori_loop(0, n_chunks // 2, pair_body, 0)
        drain(0, sem_a)
        pltpu.sync_copy(out_all, out_hbm.at[pl.ds(base, per_w)])

    return k(Yr, Gp, b)


def kernel(X, G, W, b):
    N, D = X.shape
    DEG = G.shape[1]
    U = W.shape[1]
    block = NW * C * 2
    n_pad = -(-N // block) * block

    W3 = W.reshape(DEG, D, U)
    Gp = jnp.pad(G, ((0, n_pad - N), (0, 0)))

    Zr = _tc_matmul(X, W3.astype(jnp.bfloat16), N, D, DEG, U)
    out = _sc_gather_reduce(Zr, Gp, b, n_pad, N, DEG, U)
    return out[:N]
